# baseline (speedup 1.0000x reference)
import functools

import jax
import jax.numpy as jnp
from jax import lax
from jax.experimental import pallas as pl
from jax.experimental.pallas import tpu as pltpu
from jax.experimental.pallas import tpu_sc as plsc

EMBED = 64
TOTAL = 4096 * 200
PAIRS = TOTAL // 2

_info = plsc.get_sparse_core_info()
NC, NS = _info.num_cores, _info.num_subcores
NW = NC * NS
PER_W = PAIRS // NW
CHUNK = 256
NCHUNK = PER_W // CHUNK

_mesh = plsc.VectorSubcoreMesh(core_axis_name="c", subcore_axis_name="s")


@functools.partial(
    pl.kernel,
    mesh=_mesh,
    out_type=jax.ShapeDtypeStruct((PAIRS, 2 * EMBED), jnp.float32),
    compiler_params=pltpu.CompilerParams(use_tc_tiling_on_sc=False),
    scratch_types=[
        pltpu.VMEM((NCHUNK, CHUNK), jnp.int32),
        pltpu.VMEM((NCHUNK, CHUNK), jnp.int32),
        pltpu.VMEM((CHUNK, EMBED), jnp.float32),
        pltpu.VMEM((CHUNK, EMBED), jnp.float32),
        pltpu.VMEM((CHUNK, EMBED), jnp.float32),
        pltpu.VMEM((CHUNK, EMBED), jnp.float32),
        pltpu.SemaphoreType.DMA,
        pltpu.SemaphoreType.DMA,
        pltpu.SemaphoreType.DMA,
        pltpu.SemaphoreType.DMA,
    ],
)
def _gather_kernel(idxe_hbm, idxo_hbm, table_hbm, out_hbm, idxe_v, idxo_v,
                   bufe0, bufo0, bufe1, bufo1, g0, g1, s0, s1):
    wid = lax.axis_index("s") * NC + lax.axis_index("c")
    base = wid * PER_W
    bufe = (bufe0, bufe1)
    bufo = (bufo0, bufo1)
    gsem = (g0, g1)
    ssem = (s0, s1)

    pltpu.sync_copy(idxe_hbm.at[wid], idxe_v)
    pltpu.sync_copy(idxo_hbm.at[wid], idxo_v)

    def gstart(i, b):
        pltpu.make_async_copy(table_hbm.at[idxe_v.at[i]], bufe[b],
                              gsem[b]).start()
        pltpu.make_async_copy(table_hbm.at[idxo_v.at[i]], bufo[b],
                              gsem[b]).start()

    def gwait(i, b):
        pltpu.make_async_copy(table_hbm.at[idxe_v.at[i]], bufe[b],
                              gsem[b]).wait()
        pltpu.make_async_copy(table_hbm.at[idxo_v.at[i]], bufo[b],
                              gsem[b]).wait()

    def _out_slabs(i):
        rows = pl.ds(base + i * CHUNK, CHUNK)
        return (out_hbm.at[rows, pl.ds(0, EMBED)],
                out_hbm.at[rows, pl.ds(EMBED, EMBED)])

    def sstart(i, b):
        dste, dsto = _out_slabs(i)
        pltpu.make_async_copy(bufe[b], dste, ssem[b]).start()
        pltpu.make_async_copy(bufo[b], dsto, ssem[b]).start()

    def swait(i, b):
        dste, dsto = _out_slabs(i)
        pltpu.make_async_copy(bufe[b], dste, ssem[b]).wait()
        pltpu.make_async_copy(bufo[b], dsto, ssem[b]).wait()

    gstart(0, 0)
    gstart(1, 1)
    gwait(0, 0)
    sstart(0, 0)

    def body(j, _):
        i0 = 2 * j
        swait(i0 - 2, 0)
        gstart(i0, 0)
        gwait(i0 - 1, 1)
        sstart(i0 - 1, 1)
        i1 = i0 + 1
        swait(i1 - 2, 1)
        gstart(i1, 1)
        gwait(i1 - 1, 0)
        sstart(i1 - 1, 0)
        return 0

    lax.fori_loop(1, NCHUNK // 2, body, 0)

    gwait(NCHUNK - 1, 1)
    sstart(NCHUNK - 1, 1)
    swait(NCHUNK - 2, 0)
    swait(NCHUNK - 1, 1)


def kernel(indices, embedding_weight):
    flat = indices.reshape(-1)
    idxe = flat[0::2].reshape(NW, NCHUNK, CHUNK)
    idxo = flat[1::2].reshape(NW, NCHUNK, CHUNK)
    out = _gather_kernel(idxe, idxo, embedding_weight)
    return out.reshape(indices.shape + (EMBED,))


# speedup vs baseline: 1.2037x; 1.2037x over previous
import functools

import jax
import jax.numpy as jnp
from jax import lax
from jax.experimental import pallas as pl
from jax.experimental.pallas import tpu as pltpu
from jax.experimental.pallas import tpu_sc as plsc

EMBED = 64
BATCH = 4096
HIST = 200

_info = plsc.get_sparse_core_info()
NC, NS = _info.num_cores, _info.num_subcores
NW = NC * NS
B_PER_W = BATCH // NW
NB = 2
NCHUNK = B_PER_W // NB

_mesh = plsc.VectorSubcoreMesh(core_axis_name="c", subcore_axis_name="s")


@functools.partial(
    pl.kernel,
    mesh=_mesh,
    out_type=jax.ShapeDtypeStruct((BATCH, HIST, EMBED), jnp.float32),
    compiler_params=pltpu.CompilerParams(use_tc_tiling_on_sc=False),
    scratch_types=[
        pltpu.VMEM((B_PER_W, HIST), jnp.int32),
        pltpu.VMEM((NB, HIST, EMBED), jnp.float32),
        pltpu.VMEM((NB, HIST, EMBED), jnp.float32),
        pltpu.SemaphoreType.DMA,
        pltpu.SemaphoreType.DMA,
        pltpu.SemaphoreType.DMA,
        pltpu.SemaphoreType.DMA,
    ],
)
def _gather_kernel(idx_hbm, table_hbm, out_hbm, idx_v, buf0, buf1,
                   g0, g1, s0, s1):
    wid = lax.axis_index("s") * NC + lax.axis_index("c")
    base = wid * B_PER_W
    buf = (buf0, buf1)
    gsem = (g0, g1)
    ssem = (s0, s1)

    pltpu.sync_copy(idx_hbm.at[pl.ds(base, B_PER_W)], idx_v)

    def gstart(i, b):
        for r in range(NB):
            pltpu.make_async_copy(table_hbm.at[idx_v.at[i * NB + r]],
                                  buf[b].at[r], gsem[b]).start()

    def gwait(i, b):
        for r in range(NB):
            pltpu.make_async_copy(table_hbm.at[idx_v.at[i * NB + r]],
                                  buf[b].at[r], gsem[b]).wait()

    def sstart(i, b):
        pltpu.make_async_copy(buf[b], out_hbm.at[pl.ds(base + i * NB, NB)],
                              ssem[b]).start()

    def swait(i, b):
        pltpu.make_async_copy(buf[b], out_hbm.at[pl.ds(base + i * NB, NB)],
                              ssem[b]).wait()

    gstart(0, 0)
    gstart(1, 1)
    gwait(0, 0)
    sstart(0, 0)

    def body(j, _):
        i0 = 2 * j
        swait(i0 - 2, 0)
        gstart(i0, 0)
        gwait(i0 - 1, 1)
        sstart(i0 - 1, 1)
        i1 = i0 + 1
        swait(i1 - 2, 1)
        gstart(i1, 1)
        gwait(i1 - 1, 0)
        sstart(i1 - 1, 0)
        return 0

    lax.fori_loop(1, NCHUNK // 2, body, 0)

    gwait(NCHUNK - 1, 1)
    sstart(NCHUNK - 1, 1)
    swait(NCHUNK - 2, 0)
    swait(NCHUNK - 1, 1)


def kernel(indices, embedding_weight):
    return _gather_kernel(indices, embedding_weight)
